# baseline (speedup 1.0000x reference)
import jax
import jax.numpy as jnp
from jax import lax
from jax.experimental import pallas as pl
from jax.experimental.pallas import tpu as pltpu
from jax.experimental.pallas import tpu_sc as plsc

N = 100000
B = 200000
NSAMP = 3
E = B * NSAMP
D = 128

NW = 32
P = 40
EP = P * NSAMP
NCHUNK = B // P
NITER = -(-NCHUNK // NW)
NBUF = 4
NOUTER = (NITER + 2 + NBUF - 1) // NBUF

TP = 800
TCHUNK = B // TP
TITER = -(-TCHUNK // NW)

_TC_BLK = 25000


def _mm_body(x_ref, w_ref, o_ref):
    o_ref[...] = lax.dot_general(
        x_ref[...], w_ref[...], (((1,), (1,)), ((), ())),
        preferred_element_type=jnp.float32)


def _mm_bias_relu_body(x_ref, w_ref, b_ref, o_ref):
    acc = lax.dot_general(
        x_ref[...], w_ref[...], (((1,), (1,)), ((), ())),
        preferred_element_type=jnp.float32)
    o_ref[...] = jnp.maximum(acc + b_ref[...], 0.0)


def _tc_linear(x, w):
    rows = x.shape[0]
    return pl.pallas_call(
        _mm_body,
        grid=(rows // _TC_BLK,),
        in_specs=[
            pl.BlockSpec((_TC_BLK, D), lambda i: (i, 0)),
            pl.BlockSpec((D, D), lambda i: (0, 0)),
        ],
        out_specs=pl.BlockSpec((_TC_BLK, D), lambda i: (i, 0)),
        out_shape=jax.ShapeDtypeStruct((rows, D), jnp.float32),
    )(x, w)


def _tc_linear_bias_relu(x, w, b):
    rows = x.shape[0]
    return pl.pallas_call(
        _mm_bias_relu_body,
        grid=(rows // _TC_BLK,),
        in_specs=[
            pl.BlockSpec((_TC_BLK, D), lambda i: (i, 0)),
            pl.BlockSpec((D, D), lambda i: (0, 0)),
            pl.BlockSpec((1, D), lambda i: (0, 0)),
        ],
        out_specs=pl.BlockSpec((_TC_BLK, D), lambda i: (i, 0)),
        out_shape=jax.ShapeDtypeStruct((rows, D), jnp.float32),
    )(x, w, b)


def _sc_body(g_hbm, t2_hbm, neigh_hbm, cneigh_hbm, ctgt_hbm, b1_hbm,
             out_emb_hbm, out_tgt_hbm,
             cidx_v, pidx_v, child_v, parent_v, t2_v, tin_v, trep_v, b1_v,
             isem0, isem1, isem2, isem3, gsem0, gsem1, gsem2, gsem3,
             ssem0, ssem1, ssem2, ssem3):
    isem = (isem0, isem1, isem2, isem3)
    gsem = (gsem0, gsem1, gsem2, gsem3)
    ssem = (ssem0, ssem1, ssem2, ssem3)
    wid = lax.axis_index("s") * 2 + lax.axis_index("c")

    iota16 = lax.iota(jnp.int32, 16)
    idx3 = [lax.shift_right_logical((iota16 + 16 * t) * 21846, 16)
            for t in range(NSAMP)]
    _gdn = lax.GatherDimensionNumbers(
        offset_dims=(), collapsed_slice_dims=(0,), start_index_map=(0,))

    def _vgather(x, idx):
        return lax.gather(x, idx[:, None], _gdn, (1,),
                          mode=lax.GatherScatterMode.PROMISE_IN_BOUNDS)

    def tgt_chunk(i, carry):
        c = wid + i * NW

        @pl.when(c < TCHUNK)
        def _():
            pltpu.sync_copy(ctgt_hbm.at[pl.ds(c * TP, TP)], tin_v)
            for g in range(TP // 16):
                x = tin_v[pl.ds(g * 16, 16)]
                for t in range(NSAMP):
                    trep_v[pl.ds(48 * g + 16 * t, 16)] = _vgather(x, idx3[t])
            pltpu.sync_copy(trep_v, out_tgt_hbm.at[pl.ds(c * TP * NSAMP,
                                                         TP * NSAMP)])

        return carry

    lax.fori_loop(0, TITER, tgt_chunk, 0)

    pltpu.sync_copy(b1_hbm, b1_v)
    b1k = [b1_v[pl.ds(k * 16, 16)] for k in range(D // 16)]

    def fire_idx(c, b):
        pltpu.async_copy(neigh_hbm.at[pl.ds(c * EP, EP)], cidx_v.at[b],
                         isem[b])
        pltpu.async_copy(cneigh_hbm.at[pl.ds(c * P, P)], pidx_v.at[b],
                         isem[b])

    def wait_idx(c, b):
        pltpu.make_async_copy(neigh_hbm.at[pl.ds(c * EP, EP)], cidx_v.at[b],
                              isem[b]).wait()
        pltpu.make_async_copy(cneigh_hbm.at[pl.ds(c * P, P)], pidx_v.at[b],
                              isem[b]).wait()

    def fire_gathers(c, b):
        pltpu.async_copy(g_hbm.at[cidx_v.at[b]], child_v.at[b], gsem[b])
        pltpu.async_copy(g_hbm.at[pidx_v.at[b]], parent_v.at[b], gsem[b])
        pltpu.async_copy(t2_hbm.at[pl.ds(c * P, P)], t2_v.at[b], gsem[b])

    def wait_gathers(c, b):
        pltpu.make_async_copy(g_hbm.at[cidx_v.at[b]], child_v.at[b],
                              gsem[b]).wait()
        pltpu.make_async_copy(g_hbm.at[pidx_v.at[b]], parent_v.at[b],
                              gsem[b]).wait()
        pltpu.make_async_copy(t2_hbm.at[pl.ds(c * P, P)], t2_v.at[b],
                              gsem[b]).wait()

    def fire_store(c, b):
        pltpu.async_copy(child_v.at[b], out_emb_hbm.at[pl.ds(c * EP, EP)],
                         ssem[b])

    def wait_store(c, b):
        pltpu.make_async_copy(child_v.at[b], out_emb_hbm.at[pl.ds(c * EP, EP)],
                              ssem[b]).wait()

    def compute(b):
        def path_body(p, pcarry):
            e0 = p * NSAMP
            for k in range(D // 16):
                sl = pl.ds(k * 16, 16)
                pb = b1k[k] - parent_v[b, p, sl]
                t2k = t2_v[b, p, sl]
                for j in range(NSAMP):
                    x = child_v[b, e0 + j, sl]
                    child_v[b, e0 + j, sl] = jnp.maximum(x + pb, 0.0) + t2k
            return pcarry

        lax.fori_loop(0, P, path_body, 0)

    fire_idx(wid, 0)
    wait_idx(wid, 0)
    fire_gathers(wid, 0)
    fire_idx(wid + NW, 1)

    def outer_body(i, carry):
        for b in range(NBUF):
            s = i * NBUF + b
            c_cur = wid + s * NW
            c_n1 = c_cur + NW
            c_n2 = c_cur + 2 * NW
            c_pr = c_cur - (NBUF - 1) * NW
            b1_ = (b + 1) % NBUF
            b2_ = (b + 2) % NBUF

            @pl.when((s >= NBUF - 1) & (c_pr < NCHUNK))
            def _():
                wait_store(c_pr, b1_)

            @pl.when(c_n1 < NCHUNK)
            def _():
                wait_idx(c_n1, b1_)
                fire_gathers(c_n1, b1_)

            @pl.when(c_n2 < NCHUNK)
            def _():
                fire_idx(c_n2, b2_)

            @pl.when(c_cur < NCHUNK)
            def _():
                wait_gathers(c_cur, b)
                compute(b)
                fire_store(c_cur, b)

        return carry

    lax.fori_loop(0, NOUTER, outer_body, 0)


_sc_combine = pl.kernel(
    _sc_body,
    mesh=plsc.VectorSubcoreMesh(core_axis_name="c", subcore_axis_name="s"),
    out_type=[
        jax.ShapeDtypeStruct((E, D), jnp.float32),
        jax.ShapeDtypeStruct((E,), jnp.int32),
    ],
    scratch_types=[
        pltpu.VMEM((NBUF, EP), jnp.int32),
        pltpu.VMEM((NBUF, P), jnp.int32),
        pltpu.VMEM((NBUF, EP, D), jnp.float32),
        pltpu.VMEM((NBUF, P, D), jnp.float32),
        pltpu.VMEM((NBUF, P, D), jnp.float32),
        pltpu.VMEM((TP,), jnp.int32),
        pltpu.VMEM((TP * NSAMP,), jnp.int32),
        pltpu.VMEM((D,), jnp.float32),
    ] + [pltpu.SemaphoreType.DMA] * (3 * NBUF),
)


def kernel(features, curr_embed, curr_neigh, curr_target, neigh_list,
           path_id, W1, b1, W2, b2):
    g = _tc_linear(features, W1)
    t2 = _tc_linear_bias_relu(curr_embed, W2, b2.reshape(1, D))
    new_embed, new_target = _sc_combine(
        g, t2, neigh_list, curr_neigh, curr_target, b1)
    return (neigh_list, new_embed, new_target)


# speedup vs baseline: 1.3792x; 1.3792x over previous
import jax
import jax.numpy as jnp
from jax import lax
from jax.experimental import pallas as pl
from jax.experimental.pallas import tpu as pltpu
from jax.experimental.pallas import tpu_sc as plsc

N = 100000
B = 200000
NSAMP = 3
E = B * NSAMP
D = 128

NW = 32
P = 40
EP = P * NSAMP
NCHUNK = B // P
NITER = -(-NCHUNK // NW)
NBUF = 4
NOUTER = (NITER + 2 + NBUF - 1) // NBUF

TP = 800
TCHUNK = B // TP
TITER = -(-TCHUNK // NW)

_TC_BLK = 20000


def _mm_body(x_ref, w_ref, o_ref):
    o_ref[...] = lax.dot_general(
        x_ref[...], w_ref[...], (((1,), (1,)), ((), ())),
        preferred_element_type=jnp.float32)


def _mm_bias_relu_body(x_ref, w_ref, b_ref, o_ref):
    acc = lax.dot_general(
        x_ref[...], w_ref[...], (((1,), (1,)), ((), ())),
        preferred_element_type=jnp.float32)
    o_ref[...] = jnp.maximum(acc + b_ref[...], 0.0)


def _tc_linear(x, w):
    rows = x.shape[0]
    return pl.pallas_call(
        _mm_body,
        grid=(rows // _TC_BLK,),
        in_specs=[
            pl.BlockSpec((_TC_BLK, D), lambda i: (i, 0)),
            pl.BlockSpec((D, D), lambda i: (0, 0)),
        ],
        out_specs=pl.BlockSpec((_TC_BLK, D), lambda i: (i, 0)),
        out_shape=jax.ShapeDtypeStruct((rows, D), jnp.float32),
    )(x, w)


def _tc_linear_bias_relu(x, w, b):
    rows = x.shape[0]
    return pl.pallas_call(
        _mm_bias_relu_body,
        grid=(rows // _TC_BLK,),
        in_specs=[
            pl.BlockSpec((_TC_BLK, D), lambda i: (i, 0)),
            pl.BlockSpec((D, D), lambda i: (0, 0)),
            pl.BlockSpec((1, D), lambda i: (0, 0)),
        ],
        out_specs=pl.BlockSpec((_TC_BLK, D), lambda i: (i, 0)),
        out_shape=jax.ShapeDtypeStruct((rows, D), jnp.float32),
    )(x, w, b)


def _sc_body(g_hbm, t2_hbm, neigh_hbm, cneigh_hbm, ctgt_hbm, b1_hbm,
             out_emb_hbm, out_tgt_hbm,
             cidx_v, pidx_v, child_v, parent_v, t2_v, tin_v, trep_v, b1_v,
             isem0, isem1, isem2, isem3, gsem0, gsem1, gsem2, gsem3,
             ssem0, ssem1, ssem2, ssem3):
    isem = (isem0, isem1, isem2, isem3)
    gsem = (gsem0, gsem1, gsem2, gsem3)
    ssem = (ssem0, ssem1, ssem2, ssem3)
    wid = lax.axis_index("s") * 2 + lax.axis_index("c")

    iota16 = lax.iota(jnp.int32, 16)
    idx3 = [lax.shift_right_logical((iota16 + 16 * t) * 21846, 16)
            for t in range(NSAMP)]
    _gdn = lax.GatherDimensionNumbers(
        offset_dims=(), collapsed_slice_dims=(0,), start_index_map=(0,))

    def _vgather(x, idx):
        return lax.gather(x, idx[:, None], _gdn, (1,),
                          mode=lax.GatherScatterMode.PROMISE_IN_BOUNDS)

    def tgt_chunk(i, carry):
        c = wid + i * NW

        @pl.when(c < TCHUNK)
        def _():
            pltpu.sync_copy(ctgt_hbm.at[pl.ds(c * TP, TP)], tin_v)
            for g in range(TP // 16):
                x = tin_v[pl.ds(g * 16, 16)]
                for t in range(NSAMP):
                    trep_v[pl.ds(48 * g + 16 * t, 16)] = _vgather(x, idx3[t])
            pltpu.sync_copy(trep_v, out_tgt_hbm.at[pl.ds(c * TP * NSAMP,
                                                         TP * NSAMP)])

        return carry

    lax.fori_loop(0, TITER, tgt_chunk, 0)

    pltpu.sync_copy(b1_hbm, b1_v)
    b1k = [b1_v[pl.ds(k * 16, 16)] for k in range(D // 16)]

    def fire_idx(c, b):
        pltpu.async_copy(neigh_hbm.at[pl.ds(c * EP, EP)], cidx_v.at[b],
                         isem[b])
        pltpu.async_copy(cneigh_hbm.at[pl.ds(c * P, P)], pidx_v.at[b],
                         isem[b])

    def wait_idx(c, b):
        pltpu.make_async_copy(neigh_hbm.at[pl.ds(c * EP, EP)], cidx_v.at[b],
                              isem[b]).wait()
        pltpu.make_async_copy(cneigh_hbm.at[pl.ds(c * P, P)], pidx_v.at[b],
                              isem[b]).wait()

    def fire_gathers(c, b):
        pltpu.async_copy(g_hbm.at[cidx_v.at[b]], child_v.at[b], gsem[b])
        pltpu.async_copy(g_hbm.at[pidx_v.at[b]], parent_v.at[b], gsem[b])
        pltpu.async_copy(t2_hbm.at[pl.ds(c * P, P)], t2_v.at[b], gsem[b])

    def wait_gathers(c, b):
        pltpu.make_async_copy(g_hbm.at[cidx_v.at[b]], child_v.at[b],
                              gsem[b]).wait()
        pltpu.make_async_copy(g_hbm.at[pidx_v.at[b]], parent_v.at[b],
                              gsem[b]).wait()
        pltpu.make_async_copy(t2_hbm.at[pl.ds(c * P, P)], t2_v.at[b],
                              gsem[b]).wait()

    def fire_store(c, b):
        pltpu.async_copy(child_v.at[b], out_emb_hbm.at[pl.ds(c * EP, EP)],
                         ssem[b])

    def wait_store(c, b):
        pltpu.make_async_copy(child_v.at[b], out_emb_hbm.at[pl.ds(c * EP, EP)],
                              ssem[b]).wait()

    def compute(b):
        @plsc.parallel_loop(0, P, 1, unroll=2)
        def path_body(p):
            e0 = p * NSAMP
            for k in range(D // 16):
                sl = pl.ds(k * 16, 16)
                pb = b1k[k] - parent_v[b, p, sl]
                t2k = t2_v[b, p, sl]
                for j in range(NSAMP):
                    x = child_v[b, e0 + j, sl]
                    child_v[b, e0 + j, sl] = jnp.maximum(x + pb, 0.0) + t2k

    fire_idx(wid, 0)
    wait_idx(wid, 0)
    fire_gathers(wid, 0)
    fire_idx(wid + NW, 1)

    def outer_body(i, carry):
        for b in range(NBUF):
            s = i * NBUF + b
            c_cur = wid + s * NW
            c_n1 = c_cur + NW
            c_n2 = c_cur + 2 * NW
            c_pr = c_cur - (NBUF - 1) * NW
            b1_ = (b + 1) % NBUF
            b2_ = (b + 2) % NBUF

            @pl.when((s >= NBUF - 1) & (c_pr < NCHUNK))
            def _():
                wait_store(c_pr, b1_)

            @pl.when(c_n1 < NCHUNK)
            def _():
                wait_idx(c_n1, b1_)
                fire_gathers(c_n1, b1_)

            @pl.when(c_n2 < NCHUNK)
            def _():
                fire_idx(c_n2, b2_)

            @pl.when(c_cur < NCHUNK)
            def _():
                wait_gathers(c_cur, b)
                compute(b)
                fire_store(c_cur, b)

        return carry

    lax.fori_loop(0, NOUTER, outer_body, 0)


_sc_combine = pl.kernel(
    _sc_body,
    mesh=plsc.VectorSubcoreMesh(core_axis_name="c", subcore_axis_name="s"),
    out_type=[
        jax.ShapeDtypeStruct((E, D), jnp.float32),
        jax.ShapeDtypeStruct((E,), jnp.int32),
    ],
    scratch_types=[
        pltpu.VMEM((NBUF, EP), jnp.int32),
        pltpu.VMEM((NBUF, P), jnp.int32),
        pltpu.VMEM((NBUF, EP, D), jnp.float32),
        pltpu.VMEM((NBUF, P, D), jnp.float32),
        pltpu.VMEM((NBUF, P, D), jnp.float32),
        pltpu.VMEM((TP,), jnp.int32),
        pltpu.VMEM((TP * NSAMP,), jnp.int32),
        pltpu.VMEM((D,), jnp.float32),
    ] + [pltpu.SemaphoreType.DMA] * (3 * NBUF),
)


def kernel(features, curr_embed, curr_neigh, curr_target, neigh_list,
           path_id, W1, b1, W2, b2):
    g = _tc_linear(features, W1)
    t2 = _tc_linear_bias_relu(curr_embed, W2, b2.reshape(1, D))
    new_embed, new_target = _sc_combine(
        g, t2, neigh_list, curr_neigh, curr_target, b1)
    return (neigh_list, new_embed, new_target)
